# baseline (speedup 1.0000x reference)
import functools

import jax
import jax.numpy as jnp
from jax import lax
from jax.experimental import pallas as pl
from jax.experimental.pallas import tpu as pltpu
from jax.experimental.pallas import tpu_sc as plsc

_D = 128
_BT = 4 * 8192
_T = 8192
_ND = 1024

_NC, _NS = 2, 16
_NW = _NC * _NS
_B_PER_W = _BT // _NW
_CHUNK = 128
_N_CHUNK = _B_PER_W // _CHUNK


def _sc_gather(idx_hbm, table_hbm, out_hbm, idx_v, rows_v, sem0, sem1):
    wid = lax.axis_index("s") * _NC + lax.axis_index("c")
    base = wid * _B_PER_W
    pltpu.sync_copy(idx_hbm.at[wid], idx_v)
    sems = (sem0, sem1)
    prev = pltpu.async_copy(table_hbm.at[idx_v.at[0]], rows_v.at[0], sems[0])
    for c in range(_N_CHUNK):
        if c + 1 < _N_CHUNK:
            nxt = pltpu.async_copy(
                table_hbm.at[idx_v.at[c + 1]], rows_v.at[(c + 1) % 2], sems[(c + 1) % 2]
            )
        prev.wait()
        pltpu.sync_copy(rows_v.at[c % 2], out_hbm.at[pl.ds(base + c * _CHUNK, _CHUNK)])
        if c + 1 < _N_CHUNK:
            prev = nxt


def _make_gather():
    mesh = plsc.VectorSubcoreMesh(core_axis_name="c", subcore_axis_name="s")
    return pl.kernel(
        _sc_gather,
        out_type=jax.ShapeDtypeStruct((_BT, _D), jnp.float32),
        mesh=mesh,
        scratch_types=[
            pltpu.VMEM((_N_CHUNK, _CHUNK), jnp.int32),
            pltpu.VMEM((2, _CHUNK, _D), jnp.float32),
            pltpu.SemaphoreType.DMA,
            pltpu.SemaphoreType.DMA,
        ],
    )


_TB = 1024


def _tc_matmul(g_ref, wpe_ref, w_ref, b_ref, out_ref):
    y = g_ref[...] + wpe_ref[...]
    out_ref[...] = (
        jnp.dot(y, w_ref[...], preferred_element_type=jnp.float32) + b_ref[...]
    )


def _make_matmul():
    grid = (_BT // _TB,)
    nwpe = _T // _TB
    return pl.pallas_call(
        _tc_matmul,
        grid=grid,
        in_specs=[
            pl.BlockSpec((_TB, _D), lambda i: (i, 0)),
            pl.BlockSpec((_TB, _D), lambda i: (i % nwpe, 0)),
            pl.BlockSpec((_D, _ND), lambda i: (0, 0)),
            pl.BlockSpec((1, _ND), lambda i: (0, 0)),
        ],
        out_specs=pl.BlockSpec((_TB, _ND), lambda i: (i, 0)),
        out_shape=jax.ShapeDtypeStruct((_BT, _ND), jnp.float32),
    )


@jax.jit
def kernel(x, wte, wpe, W, b):
    Bsz, T = x.shape
    idx = x.reshape(_NW, _N_CHUNK, _CHUNK)
    gathered = _make_gather()(idx, wte)
    out = _make_matmul()(gathered, wpe, W, b.reshape(1, _ND))
    return out.reshape(Bsz, T, _ND)


# speedup vs baseline: 1.0315x; 1.0315x over previous
import functools

import jax
import jax.numpy as jnp
from jax import lax
from jax.experimental import pallas as pl
from jax.experimental.pallas import tpu as pltpu
from jax.experimental.pallas import tpu_sc as plsc

_D = 128
_BT = 4 * 8192
_T = 8192
_ND = 1024

_NC, _NS = 2, 16
_NW = _NC * _NS
_B_PER_W = _BT // _NW
_CHUNK = 128
_N_CHUNK = _B_PER_W // _CHUNK


def _sc_gather(idx_hbm, table_hbm, out_hbm, idx_v, rows_v, sem0, sem1):
    wid = lax.axis_index("s") * _NC + lax.axis_index("c")
    base = wid * _B_PER_W
    pltpu.sync_copy(idx_hbm.at[wid], idx_v)
    sems = (sem0, sem1)
    prev = pltpu.async_copy(table_hbm.at[idx_v.at[0]], rows_v.at[0], sems[0])
    for c in range(_N_CHUNK):
        if c + 1 < _N_CHUNK:
            nxt = pltpu.async_copy(
                table_hbm.at[idx_v.at[c + 1]], rows_v.at[(c + 1) % 2], sems[(c + 1) % 2]
            )
        prev.wait()
        pltpu.sync_copy(rows_v.at[c % 2], out_hbm.at[pl.ds(base + c * _CHUNK, _CHUNK)])
        if c + 1 < _N_CHUNK:
            prev = nxt


def _make_gather():
    mesh = plsc.VectorSubcoreMesh(core_axis_name="c", subcore_axis_name="s")
    return pl.kernel(
        _sc_gather,
        out_type=jax.ShapeDtypeStruct((_BT, _D), jnp.float32),
        mesh=mesh,
        scratch_types=[
            pltpu.VMEM((_N_CHUNK, _CHUNK), jnp.int32),
            pltpu.VMEM((2, _CHUNK, _D), jnp.float32),
            pltpu.SemaphoreType.DMA,
            pltpu.SemaphoreType.DMA,
        ],
    )


_TB = 1024


def _tc_matmul(g_ref, wpe_ref, w_ref, b_ref, out_ref):
    y = g_ref[...] + wpe_ref[...]
    out_ref[...] = (
        jnp.dot(y, w_ref[...], preferred_element_type=jnp.float32) + b_ref[...]
    )


def _make_matmul():
    nwpe = _T // _TB
    grid = (nwpe, _BT // _T)
    return pl.pallas_call(
        _tc_matmul,
        grid=grid,
        in_specs=[
            pl.BlockSpec((_TB, _D), lambda j, k: (k * nwpe + j, 0)),
            pl.BlockSpec((_TB, _D), lambda j, k: (j, 0)),
            pl.BlockSpec((_D, _ND), lambda j, k: (0, 0)),
            pl.BlockSpec((1, _ND), lambda j, k: (0, 0)),
        ],
        out_specs=pl.BlockSpec((_TB, _ND), lambda j, k: (k * nwpe + j, 0)),
        out_shape=jax.ShapeDtypeStruct((_BT, _ND), jnp.float32),
    )


@jax.jit
def kernel(x, wte, wpe, W, b):
    Bsz, T = x.shape
    idx = x.reshape(_NW, _N_CHUNK, _CHUNK)
    gathered = _make_gather()(idx, wte)
    out = _make_matmul()(gathered, wpe, W, b.reshape(1, _ND))
    return out.reshape(Bsz, T, _ND)
